# Initial kernel scaffold; baseline (speedup 1.0000x reference)
import jax
import jax.numpy as jnp
from jax.experimental import pallas as pl


def kernel(scores, bbox_deltas, img_info, gt_boxes):
    raise NotImplementedError("write your pallas kernel here")


# speedup vs baseline: 2.3674x; 2.3674x over previous
import functools

import numpy as np
import jax
import jax.numpy as jnp
from jax.experimental import pallas as pl

_FEAT_STRIDE = 16
_PRE = 2000
_POST = 300
_THRESH = 0.7
_NPAD = 2048
_FH = _FW = 64
_B = 8


def _np_anchors(base_size=16, ratios=(0.5, 1.0, 2.0), scales=(8.0, 16.0, 32.0)):
    ratios = np.asarray(ratios)
    scales = np.asarray(scales)
    base = np.array([1.0, 1.0, float(base_size), float(base_size)]) - 1.0
    w = base[2] - base[0] + 1.0
    h = base[3] - base[1] + 1.0
    x_ctr = base[0] + 0.5 * (w - 1.0)
    y_ctr = base[1] + 0.5 * (h - 1.0)
    size = w * h
    ws_r = np.round(np.sqrt(size / ratios))
    hs_r = np.round(ws_r * ratios)

    def mk(ws, hs):
        ws = ws[:, None]
        hs = hs[:, None]
        return np.hstack(
            (
                x_ctr - 0.5 * (ws - 1.0),
                y_ctr - 0.5 * (hs - 1.0),
                x_ctr + 0.5 * (ws - 1.0),
                y_ctr + 0.5 * (hs - 1.0),
            )
        )

    ratio_anchors = mk(ws_r, hs_r)
    out = []
    for i in range(ratio_anchors.shape[0]):
        aw = ratio_anchors[i, 2] - ratio_anchors[i, 0] + 1.0
        ah = ratio_anchors[i, 3] - ratio_anchors[i, 1] + 1.0
        ax = ratio_anchors[i, 0] + 0.5 * (aw - 1.0)
        ay = ratio_anchors[i, 1] + 0.5 * (ah - 1.0)
        out.append(mk(aw * scales, ah * scales))
    return np.vstack(out)


def _np_all_anchor_stats():
    a9 = _np_anchors()
    sx = (np.arange(_FW) * _FEAT_STRIDE).astype(np.float64)
    sy = (np.arange(_FH) * _FEAT_STRIDE).astype(np.float64)
    gx, gy = np.meshgrid(sx, sy)
    shifts = np.stack([gx.ravel(), gy.ravel(), gx.ravel(), gy.ravel()], axis=1)
    allc = (a9[None, :, :] + shifts[:, None, :]).reshape(-1, 4)
    aw = allc[:, 2] - allc[:, 0] + 1.0
    ah = allc[:, 3] - allc[:, 1] + 1.0
    acx = allc[:, 0] + 0.5 * aw
    acy = allc[:, 1] + 0.5 * ah
    return (
        aw.astype(np.float32),
        ah.astype(np.float32),
        acx.astype(np.float32),
        acy.astype(np.float32),
    )


_AW, _AH, _ACX, _ACY = (jnp.asarray(v)[None, :] for v in _np_all_anchor_stats())
_NTOT = _AW.shape[1]


def _decode_body(dx, dy, dw, dh, aw, ah, acx, acy, info, ox1, oy1, ox2, oy2):
    w = aw[...]
    h = ah[...]
    pcx = dx[...] * w + acx[...]
    pcy = dy[...] * h + acy[...]
    pw = jnp.exp(dw[...]) * w
    ph = jnp.exp(dh[...]) * h
    x1 = pcx - 0.5 * pw
    y1 = pcy - 0.5 * ph
    x2 = pcx + 0.5 * pw
    y2 = pcy + 0.5 * ph
    im_h = info[:, 0:1] - 1.0
    im_w = info[:, 1:2] - 1.0
    ox1[...] = jnp.minimum(jnp.maximum(x1, 0.0), im_w)
    oy1[...] = jnp.minimum(jnp.maximum(y1, 0.0), im_h)
    ox2[...] = jnp.minimum(jnp.maximum(x2, 0.0), im_w)
    oy2[...] = jnp.minimum(jnp.maximum(y2, 0.0), im_h)


def _nms_body(x1r, y1r, x2r, y2r, keepr):
    x1 = x1r[0]
    y1 = y1r[0]
    x2 = x2r[0]
    y2 = y2r[0]
    area = (x2 - x1 + 1.0) * (y2 - y1 + 1.0)
    ridx = jax.lax.broadcasted_iota(jnp.int32, (16, 128), 0)
    cidx = jax.lax.broadcasted_iota(jnp.int32, (16, 128), 1)
    idx = ridx * 128 + cidx

    def body(i, keep):
        oh = idx == i
        xi1 = jnp.sum(jnp.where(oh, x1, 0.0))
        yi1 = jnp.sum(jnp.where(oh, y1, 0.0))
        xi2 = jnp.sum(jnp.where(oh, x2, 0.0))
        yi2 = jnp.sum(jnp.where(oh, y2, 0.0))
        ai = jnp.sum(jnp.where(oh, area, 0.0))
        ki = jnp.sum(jnp.where(oh, keep, 0.0)) > 0.0
        xx1 = jnp.maximum(xi1, x1)
        yy1 = jnp.maximum(yi1, y1)
        xx2 = jnp.minimum(xi2, x2)
        yy2 = jnp.minimum(yi2, y2)
        iw = jnp.maximum(xx2 - xx1 + 1.0, 0.0)
        ih = jnp.maximum(yy2 - yy1 + 1.0, 0.0)
        inter = iw * ih
        iou = inter / (ai + area - inter)
        sup = (iou > _THRESH) & (idx > i) & ki
        return jnp.where(sup, 0.0, keep)

    keep = jax.lax.fori_loop(0, _PRE, body, jnp.ones((16, 128), jnp.float32))
    keepr[0] = jnp.where(idx < _PRE, keep, 0.0)


@jax.jit
def _run(scores, bbox_deltas, img_info):
    scr = jnp.transpose(scores[:, 9:, :, :], (0, 2, 3, 1)).reshape(_B, -1)
    deltas = jnp.transpose(bbox_deltas, (0, 2, 3, 1)).reshape(_B, -1, 4)
    dx = deltas[..., 0]
    dy = deltas[..., 1]
    dw = deltas[..., 2]
    dh = deltas[..., 3]

    box_shape = jax.ShapeDtypeStruct((_B, _NTOT), jnp.float32)
    x1, y1, x2, y2 = pl.pallas_call(
        _decode_body,
        out_shape=(box_shape,) * 4,
    )(dx, dy, dw, dh, _AW, _AH, _ACX, _ACY, img_info)

    _, order = jax.lax.top_k(scr, _PRE)
    pad = ((0, 0), (0, _NPAD - _PRE))

    def _g(v):
        return jnp.pad(jnp.take_along_axis(v, order, axis=1), pad).reshape(
            _B, 16, 128
        )

    x1s, y1s, x2s, y2s = _g(x1), _g(y1), _g(x2), _g(y2)

    keep = pl.pallas_call(
        _nms_body,
        grid=(_B,),
        in_specs=[pl.BlockSpec((1, 16, 128), lambda b: (b, 0, 0))] * 4,
        out_specs=pl.BlockSpec((1, 16, 128), lambda b: (b, 0, 0)),
        out_shape=jax.ShapeDtypeStruct((_B, 16, 128), jnp.float32),
    )(x1s, y1s, x2s, y2s)

    keep = keep.reshape(_B, _NPAD)[:, :_PRE] > 0.5
    boxes = jnp.stack(
        [
            x1s.reshape(_B, _NPAD)[:, :_PRE],
            y1s.reshape(_B, _NPAD)[:, :_PRE],
            x2s.reshape(_B, _NPAD)[:, :_PRE],
            y2s.reshape(_B, _NPAD)[:, :_PRE],
        ],
        axis=-1,
    )

    def _compact(keep_row, box_row):
        idx = jnp.nonzero(keep_row, size=_POST, fill_value=-1)[0]
        valid = idx >= 0
        sel = jnp.take(box_row, jnp.maximum(idx, 0), axis=0)
        return jnp.where(valid[:, None], sel, 0.0)

    sel = jax.vmap(_compact)(keep, boxes)
    batch_col = jnp.broadcast_to(
        jnp.arange(_B, dtype=jnp.float32)[:, None, None], (_B, _POST, 1)
    )
    return jnp.concatenate([batch_col, sel], axis=-1)


def kernel(scores, bbox_deltas, img_info, gt_boxes):
    return _run(scores, bbox_deltas, img_info)


# speedup vs baseline: 7.7650x; 3.2800x over previous
import functools

import numpy as np
import jax
import jax.numpy as jnp
from jax.experimental import pallas as pl
from jax.experimental.pallas import tpu as pltpu

_FEAT_STRIDE = 16
_PRE = 2000
_POST = 300
_THRESH = 0.7
_NPAD = 2048
_FH = _FW = 64
_B = 8


def _np_anchors(base_size=16, ratios=(0.5, 1.0, 2.0), scales=(8.0, 16.0, 32.0)):
    ratios = np.asarray(ratios)
    scales = np.asarray(scales)
    base = np.array([1.0, 1.0, float(base_size), float(base_size)]) - 1.0
    w = base[2] - base[0] + 1.0
    h = base[3] - base[1] + 1.0
    x_ctr = base[0] + 0.5 * (w - 1.0)
    y_ctr = base[1] + 0.5 * (h - 1.0)
    size = w * h
    ws_r = np.round(np.sqrt(size / ratios))
    hs_r = np.round(ws_r * ratios)

    def mk(ws, hs):
        ws = ws[:, None]
        hs = hs[:, None]
        return np.hstack(
            (
                x_ctr - 0.5 * (ws - 1.0),
                y_ctr - 0.5 * (hs - 1.0),
                x_ctr + 0.5 * (ws - 1.0),
                y_ctr + 0.5 * (hs - 1.0),
            )
        )

    ratio_anchors = mk(ws_r, hs_r)
    out = []
    for i in range(ratio_anchors.shape[0]):
        aw = ratio_anchors[i, 2] - ratio_anchors[i, 0] + 1.0
        ah = ratio_anchors[i, 3] - ratio_anchors[i, 1] + 1.0
        ax = ratio_anchors[i, 0] + 0.5 * (aw - 1.0)
        ay = ratio_anchors[i, 1] + 0.5 * (ah - 1.0)
        out.append(mk(aw * scales, ah * scales))
    return np.vstack(out)


def _np_all_anchor_stats():
    a9 = _np_anchors()
    sx = (np.arange(_FW) * _FEAT_STRIDE).astype(np.float64)
    sy = (np.arange(_FH) * _FEAT_STRIDE).astype(np.float64)
    gx, gy = np.meshgrid(sx, sy)
    shifts = np.stack([gx.ravel(), gy.ravel(), gx.ravel(), gy.ravel()], axis=1)
    allc = (a9[None, :, :] + shifts[:, None, :]).reshape(-1, 4)
    aw = allc[:, 2] - allc[:, 0] + 1.0
    ah = allc[:, 3] - allc[:, 1] + 1.0
    acx = allc[:, 0] + 0.5 * aw
    acy = allc[:, 1] + 0.5 * ah
    return (
        aw.astype(np.float32),
        ah.astype(np.float32),
        acx.astype(np.float32),
        acy.astype(np.float32),
    )


_AW, _AH, _ACX, _ACY = (jnp.asarray(v) for v in _np_all_anchor_stats())
_NTOT = _AW.shape[0]

_POISON = 1.0e18


def _fused_body(
    dx, dy, dw, dh, aw, ah, acx, acy, imw, imh, keepr, ox1, oy1, ox2, oy2
):
    w = aw[...]
    h = ah[...]
    pcx = dx[...] * w + acx[...]
    pcy = dy[...] * h + acy[...]
    pw = jnp.exp(dw[...]) * w
    ph = jnp.exp(dh[...]) * h
    im_w = imw[...]
    im_h = imh[...]
    x1 = jnp.minimum(jnp.maximum(pcx - 0.5 * pw, 0.0), im_w)
    y1 = jnp.minimum(jnp.maximum(pcy - 0.5 * ph, 0.0), im_h)
    x2 = jnp.minimum(jnp.maximum(pcx + 0.5 * pw, 0.0), im_w)
    y2 = jnp.minimum(jnp.maximum(pcy + 0.5 * ph, 0.0), im_h)
    ox1[...] = x1
    oy1[...] = y1
    ox2[...] = x2
    oy2[...] = y2

    for r in range(16):
        nrows = 16 - r
        ridx = jax.lax.broadcasted_iota(jnp.int32, (nrows, 8, 128), 0)
        cidx = jax.lax.broadcasted_iota(jnp.int32, (nrows, 8, 128), 2)
        idxs = (ridx + r) * 128 + cidx
        y1s = y1[r:]
        y2s = y2[r:]
        cmax = 128 if r < 15 else _PRE - 15 * 128

        def body(c, carry, y1s=y1s, y2s=y2s, idxs=idxs, r=r):
            X1, X2, K = carry
            sh = jax.lax.rem(128 - c, 128)
            xi1 = pltpu.roll(X1[0], sh, 1)[:, 0:1][None]
            xi2 = pltpu.roll(X2[0], sh, 1)[:, 0:1][None]
            yi1 = pltpu.roll(y1s[0], sh, 1)[:, 0:1][None]
            yi2 = pltpu.roll(y2s[0], sh, 1)[:, 0:1][None]
            ai = (xi2 - xi1 + 1.0) * (yi2 - yi1 + 1.0)
            area = (X2 - X1 + 1.0) * (y2s - y1s + 1.0)
            iw = jnp.maximum(
                jnp.minimum(xi2, X2) - jnp.maximum(xi1, X1) + 1.0, 0.0
            )
            ih = jnp.maximum(
                jnp.minimum(yi2, y2s) - jnp.maximum(yi1, y1s) + 1.0, 0.0
            )
            inter = iw * ih
            iou = inter / (ai + area - inter)
            sup = (iou > _THRESH) & (idxs > r * 128 + c)
            return (
                jnp.where(sup, _POISON, X1),
                jnp.where(sup, -_POISON, X2),
                jnp.where(sup, 0.0, K),
            )

        if r == 0:
            carry = (x1, x2, jnp.ones((16, 8, 128), jnp.float32))
        x1c, x2c, kc = jax.lax.fori_loop(0, cmax, body, carry, unroll=4)
        keepr[r : r + 1] = jnp.where(idxs[0:1] < _PRE, kc[0:1], 0.0)
        carry = (x1c[1:], x2c[1:], kc[1:])


@jax.jit
def _run(scores, bbox_deltas, img_info):
    scr = jnp.transpose(scores[:, 9:, :, :], (0, 2, 3, 1)).reshape(_B, -1)
    deltas = jnp.transpose(bbox_deltas, (0, 2, 3, 1)).reshape(_B, -1, 4)

    _, order = jax.lax.top_k(scr, _PRE)
    pad = ((0, 0), (0, _NPAD - _PRE))

    def _g(v):
        return jnp.pad(v, pad).reshape(_B, 16, 128).transpose(1, 0, 2)

    sel_d = jnp.take_along_axis(deltas, order[..., None], axis=1)
    dxs = _g(sel_d[..., 0])
    dys = _g(sel_d[..., 1])
    dws = _g(sel_d[..., 2])
    dhs = _g(sel_d[..., 3])
    aws = _g(jnp.take(_AW, order))
    ahs = _g(jnp.take(_AH, order))
    acxs = _g(jnp.take(_ACX, order))
    acys = _g(jnp.take(_ACY, order))
    imw = (img_info[:, 1] - 1.0).reshape(1, _B, 1)
    imh = (img_info[:, 0] - 1.0).reshape(1, _B, 1)

    t = jax.ShapeDtypeStruct((16, _B, 128), jnp.float32)
    keep, x1s, y1s, x2s, y2s = pl.pallas_call(
        _fused_body,
        out_shape=(t,) * 5,
    )(dxs, dys, dws, dhs, aws, ahs, acxs, acys, imw, imh)

    def _ug(v):
        return v.transpose(1, 0, 2).reshape(_B, _NPAD)[:, :_PRE]

    keep = _ug(keep) > 0.5
    boxes = jnp.stack([_ug(x1s), _ug(y1s), _ug(x2s), _ug(y2s)], axis=-1)

    def _compact(keep_row, box_row):
        idx = jnp.nonzero(keep_row, size=_POST, fill_value=-1)[0]
        valid = idx >= 0
        sel = jnp.take(box_row, jnp.maximum(idx, 0), axis=0)
        return jnp.where(valid[:, None], sel, 0.0)

    sel = jax.vmap(_compact)(keep, boxes)
    batch_col = jnp.broadcast_to(
        jnp.arange(_B, dtype=jnp.float32)[:, None, None], (_B, _POST, 1)
    )
    return jnp.concatenate([batch_col, sel], axis=-1)


def kernel(scores, bbox_deltas, img_info, gt_boxes):
    return _run(scores, bbox_deltas, img_info)
